# baseline (speedup 1.0000x reference)
import functools

import jax
import jax.numpy as jnp
from jax import lax
from jax.experimental import pallas as pl
from jax.experimental.pallas import tpu as pltpu
from jax.experimental.pallas import tpu_sc as plsc

_ROWS = 1024
_SEQ = 26
_SEQ_PAD = 32
_NCLS = 1000
_NWORKERS = 32
_SLICES_PER_W = _ROWS // _NWORKERS
_IDX_PER_W = _SLICES_PER_W * _SEQ_PAD


def _sc_onehot(data_hbm, tmpl_hbm, rvals_hbm, out_hbm, idx_v, rvals_v,
               buf_a, buf_b, sem_a, sem_b):
    nc = 2
    wid = lax.axis_index("s") * nc + lax.axis_index("c")
    base_slice = wid * _SLICES_PER_W

    pltpu.sync_copy(data_hbm.at[pl.ds(wid * _IDX_PER_W, _IDX_PER_W)], idx_v)
    pltpu.sync_copy(rvals_hbm.at[pl.ds(wid * _IDX_PER_W, _IDX_PER_W)], rvals_v)
    pltpu.sync_copy(tmpl_hbm, buf_a)
    pltpu.sync_copy(tmpl_hbm, buf_b)

    riota = lax.iota(jnp.int32, 16)
    ones = jnp.full((16,), 1.0, jnp.float32)

    def poke(local_slice, buf):
        for g in range(2):
            cols = idx_v[pl.ds(local_slice * _SEQ_PAD + g * 16, 16)]
            rows = riota + g * 16
            mask = rows < _SEQ
            plsc.store_scatter(buf, [rows, cols], ones, mask=mask)

    def restore(local_slice, buf):
        for g in range(2):
            cols = idx_v[pl.ds(local_slice * _SEQ_PAD + g * 16, 16)]
            rows = riota + g * 16
            mask = rows < _SEQ
            vals = rvals_v[pl.ds(local_slice * _SEQ_PAD + g * 16, 16)]
            plsc.store_scatter(buf, [rows, cols], vals, mask=mask)

    def step(k, carry):
        for b, buf, sem in ((0, buf_a, sem_a), (1, buf_b, sem_b)):
            i = 2 * k + b
            g = base_slice + i

            @pl.when(i >= 2)
            def _wait_and_restore():
                pltpu.make_async_copy(buf, out_hbm.at[g - 2], sem).wait()
                restore(i - 2, buf)

            poke(i, buf)
            pltpu.async_copy(buf, out_hbm.at[g], sem)
        return carry

    lax.fori_loop(0, _SLICES_PER_W // 2, step, 0)

    last = base_slice + _SLICES_PER_W
    pltpu.make_async_copy(buf_a, out_hbm.at[last - 2], sem_a).wait()
    pltpu.make_async_copy(buf_b, out_hbm.at[last - 1], sem_b).wait()


@jax.jit
def _run(data_pad, tmpl, rvals):
    mesh = plsc.VectorSubcoreMesh(core_axis_name="c", subcore_axis_name="s")
    return pl.kernel(
        _sc_onehot,
        mesh=mesh,
        out_type=jax.ShapeDtypeStruct((_ROWS, _SEQ, _NCLS), jnp.float32),
        scratch_types=[
            pltpu.VMEM((_IDX_PER_W,), jnp.int32),
            pltpu.VMEM((_IDX_PER_W,), jnp.float32),
            pltpu.VMEM((_SEQ, _NCLS), jnp.float32),
            pltpu.VMEM((_SEQ, _NCLS), jnp.float32),
            pltpu.SemaphoreType.DMA,
            pltpu.SemaphoreType.DMA,
        ],
        compiler_params=pltpu.CompilerParams(needs_layout_passes=False),
    )(data_pad, tmpl, rvals)


def kernel(input, one_hot):
    data_pad = jnp.pad(input.astype(jnp.int32), ((0, 0), (0, _SEQ_PAD - _SEQ)))
    tmpl = jnp.tile(one_hot.astype(jnp.float32), (_SEQ, 1))
    flat = data_pad.reshape(-1)
    rvals = jnp.take(one_hot.astype(jnp.float32)[0], flat)
    return _run(flat, tmpl, rvals)
